# baseline (speedup 1.0000x reference)
import jax
import jax.numpy as jnp
from jax import lax
from jax.experimental import pallas as pl
from jax.experimental.pallas import tpu as pltpu
from jax.experimental.pallas import tpu_sc as plsc

N_NODES = 50000
BATCH = 16
N_EDGES = 1600000

NC = 2
NS = 16
LANES = 16

MICRO = 128
K = 2
CHUNK = K * MICRO
TILE_EDGES = 51200
E_PAD = NC * NS * TILE_EDGES
TILE_ROWS = TILE_EDGES // MICRO
N_CHUNKS = TILE_ROWS // K
NBUF = 2
NEBUF = 4
NP = 51200
ROWS_PER_TILE = NP // NS
ZROWS = 320
OUT_ROWS_PER_TILE = N_NODES // NS
NA = 50016
AROWS_PER_TILE = NA // NS


def _edge_kernel(aux, em, gbm, out, *scr):
    e_v = scr[0:NEBUF]
    gb_v = scr[NEBUF:2 * NEBUF]
    vv_v = scr[2 * NEBUF:2 * NEBUF + NBUF]
    zrow = scr[2 * NEBUF + NBUF]
    acc = scr[2 * NEBUF + NBUF + 1]
    aux_s = scr[2 * NEBUF + NBUF + 2]
    base = 2 * NEBUF + NBUF + 3
    csem = scr[base:base + NEBUF]
    gsem = scr[base + NEBUF:base + NEBUF + NBUF]
    ssem = scr[base + NEBUF + NBUF:base + NEBUF + 2 * NBUF]

    cid = lax.axis_index("c")
    sid = lax.axis_index("s")
    wid = sid * NC + cid
    row0 = wid * TILE_ROWS

    abase = sid * AROWS_PER_TILE
    stage = pltpu.async_copy(aux.at[pl.ds(abase, AROWS_PER_TILE)],
                             aux_s.at[pl.ds(abase, AROWS_PER_TILE)], gsem[0])

    @pl.loop(0, ZROWS)
    def _(j):
        zrow[j, :] = jnp.zeros((LANES,), jnp.float32)

    zcopies = []
    for r in range(ROWS_PER_TILE // ZROWS):
        base = sid * ROWS_PER_TILE + r * ZROWS
        zcopies.append(pltpu.async_copy(zrow, acc.at[pl.ds(base, ZROWS)], csem[0]))
    for cp in zcopies:
        cp.wait()
    stage.wait()
    plsc.subcore_barrier()

    def fire_in(c, b):
        r = row0 + c * K
        pltpu.async_copy(em.at[0, pl.ds(r, K)], e_v[b].at[pl.ds(0, K)], csem[b])
        pltpu.async_copy(em.at[1, pl.ds(r, K)], e_v[b].at[pl.ds(K, K)], csem[b])
        pltpu.async_copy(gbm.at[0, pl.ds(r, K)], gb_v[b].at[0], csem[b])
        pltpu.async_copy(gbm.at[1, pl.ds(r, K)], gb_v[b].at[1], csem[b])

    def wait_in(b):
        pltpu.make_async_copy(em.at[0, pl.ds(row0, K)], e_v[b].at[pl.ds(0, K)], csem[b]).wait()
        pltpu.make_async_copy(em.at[1, pl.ds(row0, K)], e_v[b].at[pl.ds(K, K)], csem[b]).wait()
        pltpu.make_async_copy(gbm.at[0, pl.ds(row0, K)], gb_v[b].at[0], csem[b]).wait()
        pltpu.make_async_copy(gbm.at[1, pl.ds(row0, K)], gb_v[b].at[1], csem[b]).wait()

    def fire_gather(eb, vb):
        for j in range(2 * K):
            pltpu.async_copy(aux_s.at[e_v[eb].at[j]], vv_v[vb].at[j], gsem[vb])

    def wait_gather(eb, vb):
        for j in range(2 * K):
            pltpu.make_async_copy(aux_s.at[e_v[eb].at[j]], vv_v[vb].at[j], gsem[vb]).wait()

    def fire_scatter(eb, vb):
        for j in range(2 * K):
            pltpu.async_copy(vv_v[vb].at[j], acc.at[e_v[eb].at[j]], ssem[vb], add=True)

    def wait_scatter(eb, vb):
        for j in range(2 * K):
            pltpu.make_async_copy(vv_v[vb].at[j], acc.at[e_v[eb].at[j]], ssem[vb]).wait()

    def compute(eb, vb):
        for k in range(K):
            @pl.loop(0, MICRO // LANES)
            def _(q):
                gvec = gb_v[eb][0, k, pl.ds(q * LANES, LANES)]
                bvec = gb_v[eb][1, k, pl.ds(q * LANES, LANES)]
                for l in range(LANES):
                    jj = q * LANES + l
                    vsv = vv_v[vb][k, jj, :]
                    vdv = vv_v[vb][K + k, jj, :]
                    zc = jnp.clip(gvec[l] * (vsv - vdv) + bvec[l], -20.0, 20.0)
                    e2 = jnp.exp(2.0 * zc)
                    cur = (e2 - 1.0) / (e2 + 1.0)
                    vv_v[vb][k, jj, :] = -cur
                    vv_v[vb][K + k, jj, :] = cur

    fire_in(0, 0)
    fire_in(1, 1)
    wait_in(0)
    fire_gather(0, 0)

    @pl.loop(0, N_CHUNKS // NEBUF)
    def _(cc):
        for b in range(NEBUF):
            c = cc * NEBUF + b
            vb = b % NBUF
            vbn = (b + 1) % NBUF

            @pl.when(c >= 1)
            def _():
                wait_scatter((b - 1) % NEBUF, vbn)

            @pl.when(c + 1 < N_CHUNKS)
            def _():
                wait_in((b + 1) % NEBUF)
                fire_gather((b + 1) % NEBUF, vbn)

            wait_gather(b, vb)
            compute(b, vb)
            fire_scatter(b, vb)

            @pl.when(c + 2 < N_CHUNKS)
            def _():
                fire_in(c + 2, (b + 2) % NEBUF)

    wait_scatter((N_CHUNKS - 1) % NEBUF, (N_CHUNKS - 1) % NBUF)
    plsc.subcore_barrier()

    obase = sid * OUT_ROWS_PER_TILE
    pltpu.async_copy(acc.at[pl.ds(1 + obase, OUT_ROWS_PER_TILE)],
                     out.at[cid, pl.ds(obase, OUT_ROWS_PER_TILE)], csem[0]).wait()


_edge_call = pl.kernel(
    _edge_kernel,
    out_type=jax.ShapeDtypeStruct((NC, N_NODES, LANES), jnp.float32),
    mesh=plsc.VectorSubcoreMesh(core_axis_name="c", subcore_axis_name="s"),
    compiler_params=pltpu.CompilerParams(use_tc_tiling_on_sc=False),
    scratch_types=(
        [pltpu.VMEM((2 * K, MICRO), jnp.int32) for _ in range(NEBUF)]
        + [pltpu.VMEM((2, K, MICRO), jnp.float32) for _ in range(NEBUF)]
        + [pltpu.VMEM((2 * K, MICRO, LANES), jnp.float32) for _ in range(NBUF)]
        + [pltpu.VMEM((ZROWS, LANES), jnp.float32)]
        + [pltpu.VMEM_SHARED((NP, LANES), jnp.float32)]
        + [pltpu.VMEM_SHARED((NA, LANES), jnp.float32)]
        + [pltpu.SemaphoreType.DMA for _ in range(NEBUF + 2 * NBUF)]
    ),
)


def _combine_body(p_ref, o_ref):
    o_ref[...] = jnp.transpose(p_ref[0] + p_ref[1], (1, 0))


_combine_call = pl.pallas_call(
    _combine_body,
    out_shape=jax.ShapeDtypeStruct((BATCH, N_NODES), jnp.float32),
)


def kernel(t, x, src_node, des_node, g, b):
    del t
    aux = jnp.concatenate([jnp.zeros((1, BATCH), jnp.float32), x.T,
                           jnp.zeros((NA - N_NODES - 1, BATCH), jnp.float32)], axis=0)
    pad = jnp.zeros((E_PAD - N_EDGES,), jnp.int32)
    em = jnp.concatenate([src_node, pad, des_node, pad]).reshape(2, -1, MICRO)
    fpad = jnp.zeros((E_PAD - N_EDGES,), jnp.float32)
    gbm = jnp.concatenate([g, fpad, b, fpad]).reshape(2, -1, MICRO)
    partials = _edge_call(aux, em, gbm)
    return _combine_call(partials)


# speedup vs baseline: 1.1719x; 1.1719x over previous
import jax
import jax.numpy as jnp
from jax import lax
from jax.experimental import pallas as pl
from jax.experimental.pallas import tpu as pltpu
from jax.experimental.pallas import tpu_sc as plsc

N_NODES = 50000
BATCH = 16
N_EDGES = 1600000

NC = 2
NS = 16
LANES = 16

MICRO = 128
K = 2
CHUNK = K * MICRO
TILE_EDGES = 51200
E_PAD = NC * NS * TILE_EDGES
TILE_ROWS = TILE_EDGES // MICRO
N_CHUNKS = TILE_ROWS // K
NBUF = 2
NEBUF = 4
NP = 51200
ROWS_PER_TILE = NP // NS
ZROWS = 320
OUT_ROWS_PER_TILE = N_NODES // NS
NA = 50016
AROWS_PER_TILE = NA // NS


def _edge_kernel(aux, em, gbm, out, *scr):
    e_v = scr[0:NEBUF]
    gb_v = scr[NEBUF:2 * NEBUF]
    vv_v = scr[2 * NEBUF:2 * NEBUF + NBUF]
    zrow = scr[2 * NEBUF + NBUF]
    acc = scr[2 * NEBUF + NBUF + 1]
    aux_s = scr[2 * NEBUF + NBUF + 2]
    base = 2 * NEBUF + NBUF + 3
    csem = scr[base:base + NEBUF]
    gsem = scr[base + NEBUF:base + NEBUF + NBUF]
    ssem = scr[base + NEBUF + NBUF:base + NEBUF + 2 * NBUF]

    cid = lax.axis_index("c")
    sid = lax.axis_index("s")
    wid = sid * NC + cid
    row0 = wid * TILE_ROWS

    abase = sid * AROWS_PER_TILE
    stage = pltpu.async_copy(aux.at[pl.ds(abase, AROWS_PER_TILE)],
                             aux_s.at[pl.ds(abase, AROWS_PER_TILE)], gsem[0])

    @pl.loop(0, ZROWS)
    def _(j):
        zrow[j, :] = jnp.zeros((LANES,), jnp.float32)

    zcopies = []
    for r in range(ROWS_PER_TILE // ZROWS):
        base = sid * ROWS_PER_TILE + r * ZROWS
        zcopies.append(pltpu.async_copy(zrow, acc.at[pl.ds(base, ZROWS)], csem[0]))
    for cp in zcopies:
        cp.wait()
    stage.wait()
    plsc.subcore_barrier()

    def fire_in(c, b):
        r = row0 + c * K
        pltpu.async_copy(em.at[0, pl.ds(r, K)], e_v[b].at[pl.ds(0, K)], csem[b])
        pltpu.async_copy(em.at[1, pl.ds(r, K)], e_v[b].at[pl.ds(K, K)], csem[b])
        pltpu.async_copy(gbm.at[0, pl.ds(r, K)], gb_v[b].at[0], csem[b])
        pltpu.async_copy(gbm.at[1, pl.ds(r, K)], gb_v[b].at[1], csem[b])

    def wait_in(b):
        pltpu.make_async_copy(em.at[0, pl.ds(row0, K)], e_v[b].at[pl.ds(0, K)], csem[b]).wait()
        pltpu.make_async_copy(em.at[1, pl.ds(row0, K)], e_v[b].at[pl.ds(K, K)], csem[b]).wait()
        pltpu.make_async_copy(gbm.at[0, pl.ds(row0, K)], gb_v[b].at[0], csem[b]).wait()
        pltpu.make_async_copy(gbm.at[1, pl.ds(row0, K)], gb_v[b].at[1], csem[b]).wait()

    def fire_gather(eb, vb):
        for j in range(2 * K):
            pltpu.async_copy(aux_s.at[e_v[eb].at[j]], vv_v[vb].at[j], gsem[vb])

    def wait_gather(eb, vb):
        for j in range(2 * K):
            pltpu.make_async_copy(aux_s.at[e_v[eb].at[j]], vv_v[vb].at[j], gsem[vb]).wait()

    def fire_scatter(eb, vb):
        for j in range(2 * K):
            pltpu.async_copy(vv_v[vb].at[j], acc.at[e_v[eb].at[j]], ssem[vb], add=True)

    def wait_scatter(eb, vb):
        for j in range(2 * K):
            pltpu.make_async_copy(vv_v[vb].at[j], acc.at[e_v[eb].at[j]], ssem[vb]).wait()

    def compute_k(eb, vb, k):
            @pl.loop(0, MICRO // LANES)
            def _(q):
                gvec = gb_v[eb][0, k, pl.ds(q * LANES, LANES)]
                bvec = gb_v[eb][1, k, pl.ds(q * LANES, LANES)]
                for l in range(LANES):
                    jj = q * LANES + l
                    vsv = vv_v[vb][k, jj, :]
                    vdv = vv_v[vb][K + k, jj, :]
                    zc = jnp.clip(gvec[l] * (vsv - vdv) + bvec[l], -20.0, 20.0)
                    e2 = jnp.exp(2.0 * zc)
                    cur = (e2 - 1.0) / (e2 + 1.0)
                    vv_v[vb][k, jj, :] = -cur
                    vv_v[vb][K + k, jj, :] = cur

    fire_in(0, 0)
    fire_in(1, 1)
    wait_in(0)
    fire_gather(0, 0)

    @pl.loop(0, N_CHUNKS // NEBUF)
    def _(cc):
        for b in range(NEBUF):
            c = cc * NEBUF + b
            vb = b % NBUF
            vbn = (b + 1) % NBUF

            wait_gather(b, vb)
            compute_k(b, vb, 0)

            @pl.when(c >= 1)
            def _():
                wait_scatter((b - 1) % NEBUF, vbn)

            @pl.when(c + 1 < N_CHUNKS)
            def _():
                wait_in((b + 1) % NEBUF)
                fire_gather((b + 1) % NEBUF, vbn)

            for k in range(1, K):
                compute_k(b, vb, k)
            fire_scatter(b, vb)

            @pl.when(c + 2 < N_CHUNKS)
            def _():
                fire_in(c + 2, (b + 2) % NEBUF)

    wait_scatter((N_CHUNKS - 1) % NEBUF, (N_CHUNKS - 1) % NBUF)
    plsc.subcore_barrier()

    obase = sid * OUT_ROWS_PER_TILE
    pltpu.async_copy(acc.at[pl.ds(1 + obase, OUT_ROWS_PER_TILE)],
                     out.at[cid, pl.ds(obase, OUT_ROWS_PER_TILE)], csem[0]).wait()


_edge_call = pl.kernel(
    _edge_kernel,
    out_type=jax.ShapeDtypeStruct((NC, N_NODES, LANES), jnp.float32),
    mesh=plsc.VectorSubcoreMesh(core_axis_name="c", subcore_axis_name="s"),
    compiler_params=pltpu.CompilerParams(use_tc_tiling_on_sc=False),
    scratch_types=(
        [pltpu.VMEM((2 * K, MICRO), jnp.int32) for _ in range(NEBUF)]
        + [pltpu.VMEM((2, K, MICRO), jnp.float32) for _ in range(NEBUF)]
        + [pltpu.VMEM((2 * K, MICRO, LANES), jnp.float32) for _ in range(NBUF)]
        + [pltpu.VMEM((ZROWS, LANES), jnp.float32)]
        + [pltpu.VMEM_SHARED((NP, LANES), jnp.float32)]
        + [pltpu.VMEM_SHARED((NA, LANES), jnp.float32)]
        + [pltpu.SemaphoreType.DMA for _ in range(NEBUF + 2 * NBUF)]
    ),
)


def _combine_body(p_ref, o_ref):
    o_ref[...] = jnp.transpose(p_ref[0] + p_ref[1], (1, 0))


_combine_call = pl.pallas_call(
    _combine_body,
    out_shape=jax.ShapeDtypeStruct((BATCH, N_NODES), jnp.float32),
)


def kernel(t, x, src_node, des_node, g, b):
    del t
    aux = jnp.concatenate([jnp.zeros((1, BATCH), jnp.float32), x.T,
                           jnp.zeros((NA - N_NODES - 1, BATCH), jnp.float32)], axis=0)
    pad = jnp.zeros((E_PAD - N_EDGES,), jnp.int32)
    em = jnp.concatenate([src_node, pad, des_node, pad]).reshape(2, -1, MICRO)
    fpad = jnp.zeros((E_PAD - N_EDGES,), jnp.float32)
    gbm = jnp.concatenate([g, fpad, b, fpad]).reshape(2, -1, MICRO)
    partials = _edge_call(aux, em, gbm)
    return _combine_call(partials)
